# baseline (speedup 1.0000x reference)
import functools

import jax
import jax.numpy as jnp
import numpy as np
from jax import lax
from jax.experimental import pallas as pl
from jax.experimental.pallas import tpu as pltpu
from jax.experimental.pallas import tpu_sc as plsc

INPUT_SIZE = 4096
OUTPUT_SIZE = 4096
N_GAUSS = 64
N_LOCAL = 8
N_GLOBAL = 8
TAU = 0.1
SIGMA_BOOST = 2.0
LOCAL1 = float(np.log2(INPUT_SIZE))
LOCAL2 = float(np.log2(INPUT_SIZE))
BATCH = 4096
N_POINTS = N_GAUSS * (4 + N_LOCAL + N_GLOBAL)

L = 16
N_CHUNK = N_POINTS // L
NW = 32
ROWS_PER_W = BATCH // NW


def _sample_weight(D_param, sigma_param, v_param):
    shape = jnp.array([INPUT_SIZE, OUTPUT_SIZE], dtype=jnp.float32)
    local_shape = jnp.array([LOCAL1, LOCAL2], dtype=jnp.float32)
    D = jax.nn.sigmoid(D_param) * shape
    sigma = jax.nn.softplus(sigma_param + SIGMA_BOOST)[:, None]
    sigma = jnp.repeat(sigma, 2, axis=1) * shape * 0.1 + TAU

    rkey = jax.random.key(1234)
    rk1, rk2 = jax.random.split(rkey, 2)
    select_nearest = jnp.tile(
        jnp.array([[0.0, 0.0], [0.0, 1.0], [1.0, 0.0], [1.0, 1.0]], dtype=jnp.float32),
        (N_GAUSS, 1),
    )
    select_local = (jax.random.uniform(rk1, (N_LOCAL * N_GAUSS, 2)) - 0.5) * local_shape
    part1 = jnp.repeat(D, 4, axis=0) + select_nearest
    part2 = jnp.repeat(D, N_LOCAL, axis=0) + select_local
    part3 = jax.random.uniform(rk2, (N_GLOBAL * N_GAUSS, 2)) * shape
    D_prime = jnp.concatenate([part1, part2, part3], axis=0)
    D_prime = jnp.round(D_prime)
    D_prime = jnp.stack(
        [
            jnp.clip(D_prime[:, 0], 0.0, shape[0] - 1.0),
            jnp.clip(D_prime[:, 1], 0.0, shape[1] - 1.0),
        ],
        axis=1,
    )
    D_prime = jax.lax.stop_gradient(D_prime)

    means = jnp.broadcast_to(D.T[None, :, :], (N_POINTS, 2, N_GAUSS))
    stds = jnp.broadcast_to(jnp.sqrt(sigma).T[None, :, :], (N_POINTS, 2, N_GAUSS))
    z = (D_prime[:, :, None] - means) / stds
    log_prob = -0.5 * z**2 - 0.5 * jnp.log(2.0 * jnp.pi)
    probs = jnp.exp(jnp.sum(log_prob - jnp.log(stds), axis=1))

    s = D_prime.sum(axis=1)
    cantor = s * (s + 1.0) / 2.0 + D_prime[:, 1]
    cantor_indices = jnp.argsort(cantor)
    cantor_sort = cantor[cantor_indices]
    dup = jnp.concatenate(
        [
            jnp.zeros((1,), dtype=jnp.float32),
            (cantor_sort[1:] == cantor_sort[:-1]).astype(jnp.float32),
        ]
    )
    mask = jax.lax.stop_gradient(dup[cantor_indices])

    probs = probs * (1.0 - mask)[:, None]
    probs_intermediate = probs / probs.sum(axis=0, keepdims=True)
    v_prime = (probs_intermediate * v_param).sum(axis=1)
    return D_prime.astype(jnp.int32), v_prime


R_GROUP = 2
N_BUF = 4
N_GROUP = ROWS_PER_W // R_GROUP


def _sc_body(
    x_hbm, rows_hbm, cols_hbm, vals_hbm, pos_hbm, out_hbm,
    rows_u, cols_u, vals_u, pos_u, rows_v, cols_v, vals_v,
    xb0, xb1, xb2, xb3, ob0, ob1, ob2, ob3,
    sin0, sin1, sin2, sin3, sout0, sout1, sout2, sout3,
):
    cid = lax.axis_index("c")
    sid = lax.axis_index("s")
    wid = sid * 2 + cid
    base = wid * ROWS_PER_W

    pltpu.sync_copy(rows_hbm, rows_u)
    pltpu.sync_copy(cols_hbm, cols_u)
    pltpu.sync_copy(vals_hbm, vals_u)
    pltpu.sync_copy(pos_hbm, pos_u)

    zero = jnp.zeros((L,), jnp.float32)
    lane_row = [jnp.full((L,), rr, jnp.int32) for rr in range(R_GROUP)]

    def start_in(g, xb, sin):
        pltpu.async_copy(x_hbm.at[pl.ds(base + g * R_GROUP, R_GROUP)], xb, sin)

    def wait_in(xb, sin):
        pltpu.make_async_copy(x_hbm.at[pl.ds(base, R_GROUP)], xb, sin).wait()

    def start_out(g, ob, sout):
        pltpu.async_copy(ob, out_hbm.at[pl.ds(base + g * R_GROUP, R_GROUP)], sout)

    def wait_out(ob, sout):
        pltpu.make_async_copy(ob, out_hbm.at[pl.ds(base, R_GROUP)], sout).wait()

    def accumulate(xb, ob):
        def chunk_body(j, c):
            r = rows_v[pl.ds(j * L, L)]
            cc = cols_v[pl.ds(j * L, L)]
            vv = vals_v[pl.ds(j * L, L)]
            for rr in range(R_GROUP):
                xe = plsc.load_gather(xb, [lane_row[rr], r])
                plsc.addupdate_scatter(ob, [lane_row[rr], cc], xe * vv)
            return c

        lax.fori_loop(0, N_CHUNK, chunk_body, 0, unroll=2)

    def cleanup(ob):
        def chunk_body(j, c):
            cc = cols_v[pl.ds(j * L, L)]
            for rr in range(R_GROUP):
                plsc.store_scatter(ob, [lane_row[rr], cc], zero)
            return c

        lax.fori_loop(0, N_CHUNK, chunk_body, 0, unroll=2)

    bufs = (
        (xb0, ob0, sin0, sout0),
        (xb1, ob1, sin1, sout1),
        (xb2, ob2, sin2, sout2),
        (xb3, ob3, sin3, sout3),
    )

    for b, (xb, _, sin, _) in enumerate(bufs):
        start_in(b, xb, sin)

    def permute_body(j, c):
        sl = pl.ds(j * L, L)
        p = pos_u[sl]
        slot = (p % N_CHUNK) * L + p // N_CHUNK
        plsc.store_scatter(rows_v, [slot], rows_u[sl])
        plsc.store_scatter(cols_v, [slot], cols_u[sl])
        plsc.store_scatter(vals_v, [slot], vals_u[sl])
        return c

    lax.fori_loop(0, N_CHUNK, permute_body, 0, unroll=2)

    def zero_body(j, c):
        for _, ob, _, _ in bufs:
            for rr in range(R_GROUP):
                ob[rr, pl.ds(j * L, L)] = zero
        return c

    lax.fori_loop(0, OUTPUT_SIZE // L, zero_body, 0, unroll=4)

    def ring_body(h, carry):
        for b, (xb, ob, sin, sout) in enumerate(bufs):
            g = N_BUF * h + b

            @pl.when(h >= 1)
            def _():
                wait_out(ob, sout)
                cleanup(ob)

            wait_in(xb, sin)
            accumulate(xb, ob)
            start_out(g, ob, sout)

            @pl.when(h < N_GROUP // N_BUF - 1)
            def _():
                start_in(g + N_BUF, xb, sin)

        return carry

    lax.fori_loop(0, N_GROUP // N_BUF, ring_body, 0)

    for _, ob, _, sout in bufs:
        wait_out(ob, sout)


@jax.jit
def _sparse_mm_sc(x, rows, cols, vals, pos):
    kern = pl.kernel(
        _sc_body,
        out_type=jax.ShapeDtypeStruct((BATCH, OUTPUT_SIZE), jnp.float32),
        mesh=plsc.VectorSubcoreMesh(core_axis_name="c", subcore_axis_name="s"),
        scratch_types=[
            pltpu.VMEM((N_POINTS,), jnp.int32),
            pltpu.VMEM((N_POINTS,), jnp.int32),
            pltpu.VMEM((N_POINTS,), jnp.float32),
            pltpu.VMEM((N_POINTS,), jnp.int32),
            pltpu.VMEM((N_POINTS,), jnp.int32),
            pltpu.VMEM((N_POINTS,), jnp.int32),
            pltpu.VMEM((N_POINTS,), jnp.float32),
            *[pltpu.VMEM((R_GROUP, INPUT_SIZE), jnp.float32) for _ in range(N_BUF)],
            *[pltpu.VMEM((R_GROUP, OUTPUT_SIZE), jnp.float32) for _ in range(N_BUF)],
            *[pltpu.SemaphoreType.DMA for _ in range(2 * N_BUF)],
        ],
        compiler_params=pltpu.CompilerParams(needs_layout_passes=False),
    )
    return kern(x, rows, cols, vals, pos)


def kernel(x, D, sigma, v):
    indices, values = _sample_weight(D, sigma, v)
    rows = indices[:, 0]
    cols = indices[:, 1]

    iota = jnp.arange(N_POINTS, dtype=jnp.int32)
    key = cols * jnp.int32(2048) + iota
    pos = jnp.sum((key[None, :] < key[:, None]).astype(jnp.int32), axis=1)

    return _sparse_mm_sc(x, rows, cols, values, pos)


# speedup vs baseline: 1.2055x; 1.2055x over previous
import functools

import jax
import jax.numpy as jnp
import numpy as np
from jax import lax
from jax.experimental import pallas as pl
from jax.experimental.pallas import tpu as pltpu
from jax.experimental.pallas import tpu_sc as plsc

INPUT_SIZE = 4096
OUTPUT_SIZE = 4096
N_GAUSS = 64
N_LOCAL = 8
N_GLOBAL = 8
TAU = 0.1
SIGMA_BOOST = 2.0
LOCAL1 = float(np.log2(INPUT_SIZE))
LOCAL2 = float(np.log2(INPUT_SIZE))
BATCH = 4096
N_POINTS = N_GAUSS * (4 + N_LOCAL + N_GLOBAL)

L = 16
N_CHUNK = N_POINTS // L
NW = 32
ROWS_PER_W = BATCH // NW


def _sample_weight(D_param, sigma_param, v_param):
    shape = jnp.array([INPUT_SIZE, OUTPUT_SIZE], dtype=jnp.float32)
    local_shape = jnp.array([LOCAL1, LOCAL2], dtype=jnp.float32)
    D = jax.nn.sigmoid(D_param) * shape
    sigma = jax.nn.softplus(sigma_param + SIGMA_BOOST)[:, None]
    sigma = jnp.repeat(sigma, 2, axis=1) * shape * 0.1 + TAU

    rkey = jax.random.key(1234)
    rk1, rk2 = jax.random.split(rkey, 2)
    select_nearest = jnp.tile(
        jnp.array([[0.0, 0.0], [0.0, 1.0], [1.0, 0.0], [1.0, 1.0]], dtype=jnp.float32),
        (N_GAUSS, 1),
    )
    select_local = (jax.random.uniform(rk1, (N_LOCAL * N_GAUSS, 2)) - 0.5) * local_shape
    part1 = jnp.repeat(D, 4, axis=0) + select_nearest
    part2 = jnp.repeat(D, N_LOCAL, axis=0) + select_local
    part3 = jax.random.uniform(rk2, (N_GLOBAL * N_GAUSS, 2)) * shape
    D_prime = jnp.concatenate([part1, part2, part3], axis=0)
    D_prime = jnp.round(D_prime)
    D_prime = jnp.stack(
        [
            jnp.clip(D_prime[:, 0], 0.0, shape[0] - 1.0),
            jnp.clip(D_prime[:, 1], 0.0, shape[1] - 1.0),
        ],
        axis=1,
    )
    D_prime = jax.lax.stop_gradient(D_prime)

    means = jnp.broadcast_to(D.T[None, :, :], (N_POINTS, 2, N_GAUSS))
    stds = jnp.broadcast_to(jnp.sqrt(sigma).T[None, :, :], (N_POINTS, 2, N_GAUSS))
    z = (D_prime[:, :, None] - means) / stds
    log_prob = -0.5 * z**2 - 0.5 * jnp.log(2.0 * jnp.pi)
    probs = jnp.exp(jnp.sum(log_prob - jnp.log(stds), axis=1))

    s = D_prime.sum(axis=1)
    cantor = s * (s + 1.0) / 2.0 + D_prime[:, 1]
    cantor_indices = jnp.argsort(cantor)
    cantor_sort = cantor[cantor_indices]
    dup = jnp.concatenate(
        [
            jnp.zeros((1,), dtype=jnp.float32),
            (cantor_sort[1:] == cantor_sort[:-1]).astype(jnp.float32),
        ]
    )
    mask = jax.lax.stop_gradient(dup[cantor_indices])

    probs = probs * (1.0 - mask)[:, None]
    probs_intermediate = probs / probs.sum(axis=0, keepdims=True)
    v_prime = (probs_intermediate * v_param).sum(axis=1)
    return D_prime.astype(jnp.int32), v_prime


R_GROUP = 4
N_IN_BUF = 4
N_OUT_BUF = 2
N_GROUP = ROWS_PER_W // R_GROUP


def _sc_body(
    x_hbm, rows_hbm, cols_hbm, vals_hbm, pos_hbm, out_hbm,
    rows_u, cols_u, vals_u, pos_u, rows_v, cols_v, vals_v,
    xb0, xb1, xb2, xb3, ob0, ob1,
    sin0, sin1, sin2, sin3, sout0, sout1,
):
    cid = lax.axis_index("c")
    sid = lax.axis_index("s")
    wid = sid * 2 + cid
    base = wid * ROWS_PER_W

    pltpu.sync_copy(rows_hbm, rows_u)
    pltpu.sync_copy(cols_hbm, cols_u)
    pltpu.sync_copy(vals_hbm, vals_u)
    pltpu.sync_copy(pos_hbm, pos_u)

    zero = jnp.zeros((L,), jnp.float32)
    lane_row = [jnp.full((L,), rr, jnp.int32) for rr in range(R_GROUP)]

    def start_in(g, xb, sin):
        pltpu.async_copy(x_hbm.at[pl.ds(base + g * R_GROUP, R_GROUP)], xb, sin)

    def wait_in(xb, sin):
        pltpu.make_async_copy(x_hbm.at[pl.ds(base, R_GROUP)], xb, sin).wait()

    def start_out(g, ob, sout):
        pltpu.async_copy(ob, out_hbm.at[pl.ds(base + g * R_GROUP, R_GROUP)], sout)

    def wait_out(ob, sout):
        pltpu.make_async_copy(ob, out_hbm.at[pl.ds(base, R_GROUP)], sout).wait()

    def accumulate(xb, ob):
        def chunk_body(j, c):
            r = rows_v[pl.ds(j * L, L)]
            cc = cols_v[pl.ds(j * L, L)]
            vv = vals_v[pl.ds(j * L, L)]
            for rr in range(R_GROUP):
                xe = plsc.load_gather(xb, [lane_row[rr], r])
                plsc.addupdate_scatter(ob, [lane_row[rr], cc], xe * vv)
            return c

        lax.fori_loop(0, N_CHUNK, chunk_body, 0, unroll=2)

    def cleanup(ob):
        def chunk_body(j, c):
            cc = cols_v[pl.ds(j * L, L)]
            for rr in range(R_GROUP):
                plsc.store_scatter(ob, [lane_row[rr], cc], zero)
            return c

        lax.fori_loop(0, N_CHUNK, chunk_body, 0, unroll=2)

    in_bufs = ((xb0, sin0), (xb1, sin1), (xb2, sin2), (xb3, sin3))
    out_bufs = ((ob0, sout0), (ob1, sout1))

    for b, (xb, sin) in enumerate(in_bufs):
        start_in(b, xb, sin)

    def permute_body(j, c):
        sl = pl.ds(j * L, L)
        p = pos_u[sl]
        slot = (p % N_CHUNK) * L + p // N_CHUNK
        plsc.store_scatter(rows_v, [slot], rows_u[sl])
        plsc.store_scatter(cols_v, [slot], cols_u[sl])
        plsc.store_scatter(vals_v, [slot], vals_u[sl])
        return c

    lax.fori_loop(0, N_CHUNK, permute_body, 0, unroll=2)

    def zero_body(j, c):
        for ob, _ in out_bufs:
            for rr in range(R_GROUP):
                ob[rr, pl.ds(j * L, L)] = zero
        return c

    lax.fori_loop(0, OUTPUT_SIZE // L, zero_body, 0, unroll=4)

    def ring_body(h, carry):
        for q in range(N_IN_BUF):
            g = N_IN_BUF * h + q
            xb, sin = in_bufs[q]
            ob, sout = out_bufs[q % N_OUT_BUF]

            @pl.when(g >= N_OUT_BUF)
            def _():
                wait_out(ob, sout)
                cleanup(ob)

            wait_in(xb, sin)
            accumulate(xb, ob)
            start_out(g, ob, sout)

            @pl.when(g + N_IN_BUF <= N_GROUP - 1)
            def _():
                start_in(g + N_IN_BUF, xb, sin)

        return carry

    lax.fori_loop(0, N_GROUP // N_IN_BUF, ring_body, 0)

    for ob, sout in out_bufs:
        wait_out(ob, sout)


@jax.jit
def _sparse_mm_sc(x, rows, cols, vals, pos):
    kern = pl.kernel(
        _sc_body,
        out_type=jax.ShapeDtypeStruct((BATCH, OUTPUT_SIZE), jnp.float32),
        mesh=plsc.VectorSubcoreMesh(core_axis_name="c", subcore_axis_name="s"),
        scratch_types=[
            pltpu.VMEM((N_POINTS,), jnp.int32),
            pltpu.VMEM((N_POINTS,), jnp.int32),
            pltpu.VMEM((N_POINTS,), jnp.float32),
            pltpu.VMEM((N_POINTS,), jnp.int32),
            pltpu.VMEM((N_POINTS,), jnp.int32),
            pltpu.VMEM((N_POINTS,), jnp.int32),
            pltpu.VMEM((N_POINTS,), jnp.float32),
            *[pltpu.VMEM((R_GROUP, INPUT_SIZE), jnp.float32) for _ in range(N_IN_BUF)],
            *[pltpu.VMEM((R_GROUP, OUTPUT_SIZE), jnp.float32) for _ in range(N_OUT_BUF)],
            *[pltpu.SemaphoreType.DMA for _ in range(N_IN_BUF + N_OUT_BUF)],
        ],
        compiler_params=pltpu.CompilerParams(needs_layout_passes=False),
    )
    return kern(x, rows, cols, vals, pos)


def kernel(x, D, sigma, v):
    indices, values = _sample_weight(D, sigma, v)
    rows = indices[:, 0]
    cols = indices[:, 1]

    iota = jnp.arange(N_POINTS, dtype=jnp.int32)
    key = cols * jnp.int32(2048) + iota
    pos = jnp.sum((key[None, :] < key[:, None]).astype(jnp.int32), axis=1)

    return _sparse_mm_sc(x, rows, cols, values, pos)


# speedup vs baseline: 1.9937x; 1.6538x over previous
import functools

import jax
import jax.numpy as jnp
import numpy as np
from jax import lax
from jax.experimental import pallas as pl
from jax.experimental.pallas import tpu as pltpu
from jax.experimental.pallas import tpu_sc as plsc

INPUT_SIZE = 4096
OUTPUT_SIZE = 4096
N_GAUSS = 64
N_LOCAL = 8
N_GLOBAL = 8
TAU = 0.1
SIGMA_BOOST = 2.0
LOCAL1 = float(np.log2(INPUT_SIZE))
LOCAL2 = float(np.log2(INPUT_SIZE))
BATCH = 4096
N_POINTS = N_GAUSS * (4 + N_LOCAL + N_GLOBAL)

L = 16
N_CHUNK = N_POINTS // L
NW = 32
ROWS_PER_W = BATCH // NW


def _sample_weight(D_param, sigma_param, v_param):
    shape = jnp.array([INPUT_SIZE, OUTPUT_SIZE], dtype=jnp.float32)
    local_shape = jnp.array([LOCAL1, LOCAL2], dtype=jnp.float32)
    D = jax.nn.sigmoid(D_param) * shape
    sigma = jax.nn.softplus(sigma_param + SIGMA_BOOST)[:, None]
    sigma = jnp.repeat(sigma, 2, axis=1) * shape * 0.1 + TAU

    rkey = jax.random.key(1234)
    rk1, rk2 = jax.random.split(rkey, 2)
    select_nearest = jnp.tile(
        jnp.array([[0.0, 0.0], [0.0, 1.0], [1.0, 0.0], [1.0, 1.0]], dtype=jnp.float32),
        (N_GAUSS, 1),
    )
    select_local = (jax.random.uniform(rk1, (N_LOCAL * N_GAUSS, 2)) - 0.5) * local_shape
    part1 = jnp.repeat(D, 4, axis=0) + select_nearest
    part2 = jnp.repeat(D, N_LOCAL, axis=0) + select_local
    part3 = jax.random.uniform(rk2, (N_GLOBAL * N_GAUSS, 2)) * shape
    D_prime = jnp.concatenate([part1, part2, part3], axis=0)
    D_prime = jnp.round(D_prime)
    D_prime = jnp.stack(
        [
            jnp.clip(D_prime[:, 0], 0.0, shape[0] - 1.0),
            jnp.clip(D_prime[:, 1], 0.0, shape[1] - 1.0),
        ],
        axis=1,
    )
    D_prime = jax.lax.stop_gradient(D_prime)

    means = jnp.broadcast_to(D.T[None, :, :], (N_POINTS, 2, N_GAUSS))
    stds = jnp.broadcast_to(jnp.sqrt(sigma).T[None, :, :], (N_POINTS, 2, N_GAUSS))
    z = (D_prime[:, :, None] - means) / stds
    log_prob = -0.5 * z**2 - 0.5 * jnp.log(2.0 * jnp.pi)
    probs = jnp.exp(jnp.sum(log_prob - jnp.log(stds), axis=1))

    s = D_prime.sum(axis=1)
    cantor = s * (s + 1.0) / 2.0 + D_prime[:, 1]
    cantor_indices = jnp.argsort(cantor)
    cantor_sort = cantor[cantor_indices]
    dup = jnp.concatenate(
        [
            jnp.zeros((1,), dtype=jnp.float32),
            (cantor_sort[1:] == cantor_sort[:-1]).astype(jnp.float32),
        ]
    )
    mask = jax.lax.stop_gradient(dup[cantor_indices])

    probs = probs * (1.0 - mask)[:, None]
    probs_intermediate = probs / probs.sum(axis=0, keepdims=True)
    v_prime = (probs_intermediate * v_param).sum(axis=1)
    return D_prime.astype(jnp.int32), v_prime


R_GROUP = 4
N_IN_BUF = 4
N_OUT_BUF = 2
N_GROUP = ROWS_PER_W // R_GROUP


def _sc_body(
    x_hbm, rows_hbm, cols_hbm, vals_hbm, pos_hbm, out_hbm,
    rows_u, cols_u, vals_u, pos_u, rows_v, cols_v, vals_v,
    xb0, xb1, xb2, xb3, ob0, ob1,
    sin0, sin1, sin2, sin3, sout0, sout1,
):
    cid = lax.axis_index("c")
    sid = lax.axis_index("s")
    wid = sid * 2 + cid
    base = wid * ROWS_PER_W

    pltpu.sync_copy(rows_hbm, rows_u)
    pltpu.sync_copy(cols_hbm, cols_u)
    pltpu.sync_copy(vals_hbm, vals_u)
    pltpu.sync_copy(pos_hbm, pos_u)

    zero = jnp.zeros((L,), jnp.float32)
    lane_row = [jnp.full((L,), rr, jnp.int32) for rr in range(R_GROUP)]

    def start_in(g, xb, sin):
        pltpu.async_copy(x_hbm.at[pl.ds(base + g * R_GROUP, R_GROUP)], xb, sin)

    def wait_in(xb, sin):
        pltpu.make_async_copy(x_hbm.at[pl.ds(base, R_GROUP)], xb, sin).wait()

    def start_out(g, ob, sout):
        pltpu.async_copy(ob, out_hbm.at[pl.ds(base + g * R_GROUP, R_GROUP)], sout)

    def wait_out(ob, sout):
        pltpu.make_async_copy(ob, out_hbm.at[pl.ds(base, R_GROUP)], sout).wait()

    def accumulate(xb, ob):
        @plsc.parallel_loop(0, N_CHUNK, unroll=4)
        def chunk_body(j):
            r = rows_v[pl.ds(j * L, L)]
            cc = cols_v[pl.ds(j * L, L)]
            vv = vals_v[pl.ds(j * L, L)]
            for rr in range(R_GROUP):
                xe = plsc.load_gather(xb, [lane_row[rr], r])
                plsc.addupdate_scatter(ob, [lane_row[rr], cc], xe * vv)

    def cleanup(ob):
        @plsc.parallel_loop(0, N_CHUNK, unroll=4)
        def chunk_body(j):
            cc = cols_v[pl.ds(j * L, L)]
            for rr in range(R_GROUP):
                plsc.store_scatter(ob, [lane_row[rr], cc], zero)

    in_bufs = ((xb0, sin0), (xb1, sin1), (xb2, sin2), (xb3, sin3))
    out_bufs = ((ob0, sout0), (ob1, sout1))

    for b, (xb, sin) in enumerate(in_bufs):
        start_in(b, xb, sin)

    @plsc.parallel_loop(0, N_CHUNK, unroll=2)
    def permute_body(j):
        sl = pl.ds(j * L, L)
        p = pos_u[sl]
        slot = (p % N_CHUNK) * L + p // N_CHUNK
        plsc.store_scatter(rows_v, [slot], rows_u[sl])
        plsc.store_scatter(cols_v, [slot], cols_u[sl])
        plsc.store_scatter(vals_v, [slot], vals_u[sl])

    @plsc.parallel_loop(0, OUTPUT_SIZE // L, unroll=4)
    def zero_body(j):
        for ob, _ in out_bufs:
            for rr in range(R_GROUP):
                ob[rr, pl.ds(j * L, L)] = zero

    def ring_body(h, carry):
        for q in range(N_IN_BUF):
            g = N_IN_BUF * h + q
            xb, sin = in_bufs[q]
            ob, sout = out_bufs[q % N_OUT_BUF]

            @pl.when(g >= N_OUT_BUF)
            def _():
                wait_out(ob, sout)
                cleanup(ob)

            wait_in(xb, sin)
            accumulate(xb, ob)
            start_out(g, ob, sout)

            @pl.when(g + N_IN_BUF <= N_GROUP - 1)
            def _():
                start_in(g + N_IN_BUF, xb, sin)

        return carry

    lax.fori_loop(0, N_GROUP // N_IN_BUF, ring_body, 0)

    for ob, sout in out_bufs:
        wait_out(ob, sout)


@jax.jit
def _sparse_mm_sc(x, rows, cols, vals, pos):
    kern = pl.kernel(
        _sc_body,
        out_type=jax.ShapeDtypeStruct((BATCH, OUTPUT_SIZE), jnp.float32),
        mesh=plsc.VectorSubcoreMesh(core_axis_name="c", subcore_axis_name="s"),
        scratch_types=[
            pltpu.VMEM((N_POINTS,), jnp.int32),
            pltpu.VMEM((N_POINTS,), jnp.int32),
            pltpu.VMEM((N_POINTS,), jnp.float32),
            pltpu.VMEM((N_POINTS,), jnp.int32),
            pltpu.VMEM((N_POINTS,), jnp.int32),
            pltpu.VMEM((N_POINTS,), jnp.int32),
            pltpu.VMEM((N_POINTS,), jnp.float32),
            *[pltpu.VMEM((R_GROUP, INPUT_SIZE), jnp.float32) for _ in range(N_IN_BUF)],
            *[pltpu.VMEM((R_GROUP, OUTPUT_SIZE), jnp.float32) for _ in range(N_OUT_BUF)],
            *[pltpu.SemaphoreType.DMA for _ in range(N_IN_BUF + N_OUT_BUF)],
        ],
        compiler_params=pltpu.CompilerParams(needs_layout_passes=False),
    )
    return kern(x, rows, cols, vals, pos)


def kernel(x, D, sigma, v):
    indices, values = _sample_weight(D, sigma, v)
    rows = indices[:, 0]
    cols = indices[:, 1]

    iota = jnp.arange(N_POINTS, dtype=jnp.int32)
    key = cols * jnp.int32(2048) + iota
    pos = jnp.sum((key[None, :] < key[:, None]).astype(jnp.int32), axis=1)

    return _sparse_mm_sc(x, rows, cols, values, pos)


# speedup vs baseline: 2.2470x; 1.1270x over previous
import functools

import jax
import jax.numpy as jnp
import numpy as np
from jax import lax
from jax.experimental import pallas as pl
from jax.experimental.pallas import tpu as pltpu
from jax.experimental.pallas import tpu_sc as plsc

INPUT_SIZE = 4096
OUTPUT_SIZE = 4096
N_GAUSS = 64
N_LOCAL = 8
N_GLOBAL = 8
TAU = 0.1
SIGMA_BOOST = 2.0
LOCAL1 = float(np.log2(INPUT_SIZE))
LOCAL2 = float(np.log2(INPUT_SIZE))
BATCH = 4096
N_POINTS = N_GAUSS * (4 + N_LOCAL + N_GLOBAL)

L = 16
N_CHUNK = N_POINTS // L
NW = 32
ROWS_PER_W = BATCH // NW


def _sample_weight(D_param, sigma_param, v_param):
    shape = jnp.array([INPUT_SIZE, OUTPUT_SIZE], dtype=jnp.float32)
    local_shape = jnp.array([LOCAL1, LOCAL2], dtype=jnp.float32)
    D = jax.nn.sigmoid(D_param) * shape
    sigma = jax.nn.softplus(sigma_param + SIGMA_BOOST)[:, None]
    sigma = jnp.repeat(sigma, 2, axis=1) * shape * 0.1 + TAU

    rkey = jax.random.key(1234)
    rk1, rk2 = jax.random.split(rkey, 2)
    select_nearest = jnp.tile(
        jnp.array([[0.0, 0.0], [0.0, 1.0], [1.0, 0.0], [1.0, 1.0]], dtype=jnp.float32),
        (N_GAUSS, 1),
    )
    select_local = (jax.random.uniform(rk1, (N_LOCAL * N_GAUSS, 2)) - 0.5) * local_shape
    part1 = jnp.repeat(D, 4, axis=0) + select_nearest
    part2 = jnp.repeat(D, N_LOCAL, axis=0) + select_local
    part3 = jax.random.uniform(rk2, (N_GLOBAL * N_GAUSS, 2)) * shape
    D_prime = jnp.concatenate([part1, part2, part3], axis=0)
    D_prime = jnp.round(D_prime)
    D_prime = jnp.stack(
        [
            jnp.clip(D_prime[:, 0], 0.0, shape[0] - 1.0),
            jnp.clip(D_prime[:, 1], 0.0, shape[1] - 1.0),
        ],
        axis=1,
    )
    D_prime = jax.lax.stop_gradient(D_prime)

    means = jnp.broadcast_to(D.T[None, :, :], (N_POINTS, 2, N_GAUSS))
    stds = jnp.broadcast_to(jnp.sqrt(sigma).T[None, :, :], (N_POINTS, 2, N_GAUSS))
    z = (D_prime[:, :, None] - means) / stds
    log_prob = -0.5 * z**2 - 0.5 * jnp.log(2.0 * jnp.pi)
    probs = jnp.exp(jnp.sum(log_prob - jnp.log(stds), axis=1))

    s = D_prime.sum(axis=1)
    cantor = s * (s + 1.0) / 2.0 + D_prime[:, 1]
    iota = jnp.arange(N_POINTS, dtype=jnp.int32)
    lt = cantor[None, :] < cantor[:, None]
    eqlow = (cantor[None, :] == cantor[:, None]) & (iota[None, :] < iota[:, None])
    rank = jnp.sum((lt | eqlow).astype(jnp.int32), axis=1)
    nf = (jnp.sum(eqlow.astype(jnp.int32), axis=1) > 0).astype(jnp.float32)
    perm = jnp.sum(jnp.where(rank[None, :] == iota[:, None], iota[None, :], 0), axis=1)
    perm2 = jnp.sum(jnp.where(rank[None, :] == perm[:, None], iota[None, :], 0), axis=1)
    mask = jnp.sum(jnp.where(iota[None, :] == perm2[:, None], nf[None, :], 0.0), axis=1)

    probs = probs * (1.0 - mask)[:, None]
    probs_intermediate = probs / probs.sum(axis=0, keepdims=True)
    v_prime = (probs_intermediate * v_param).sum(axis=1)
    return D_prime.astype(jnp.int32), v_prime


R_GROUP = 4
N_IN_BUF = 4
N_OUT_BUF = 2
N_GROUP = ROWS_PER_W // R_GROUP


def _sc_body(
    x_hbm, rows_hbm, cols_hbm, vals_hbm, pos_hbm, out_hbm,
    rows_u, cols_u, vals_u, pos_u, rows_v, cols_v, vals_v,
    xb0, xb1, xb2, xb3, ob0, ob1,
    sin0, sin1, sin2, sin3, sout0, sout1,
):
    cid = lax.axis_index("c")
    sid = lax.axis_index("s")
    wid = sid * 2 + cid
    base = wid * ROWS_PER_W

    pltpu.sync_copy(rows_hbm, rows_u)
    pltpu.sync_copy(cols_hbm, cols_u)
    pltpu.sync_copy(vals_hbm, vals_u)
    pltpu.sync_copy(pos_hbm, pos_u)

    zero = jnp.zeros((L,), jnp.float32)
    lane_row = [jnp.full((L,), rr, jnp.int32) for rr in range(R_GROUP)]

    def start_in(g, xb, sin):
        pltpu.async_copy(x_hbm.at[pl.ds(base + g * R_GROUP, R_GROUP)], xb, sin)

    def wait_in(xb, sin):
        pltpu.make_async_copy(x_hbm.at[pl.ds(base, R_GROUP)], xb, sin).wait()

    def start_out(g, ob, sout):
        pltpu.async_copy(ob, out_hbm.at[pl.ds(base + g * R_GROUP, R_GROUP)], sout)

    def wait_out(ob, sout):
        pltpu.make_async_copy(ob, out_hbm.at[pl.ds(base, R_GROUP)], sout).wait()

    def accumulate(xb, ob):
        @plsc.parallel_loop(0, N_CHUNK, unroll=4)
        def chunk_body(j):
            r = rows_v[pl.ds(j * L, L)]
            cc = cols_v[pl.ds(j * L, L)]
            vv = vals_v[pl.ds(j * L, L)]
            for rr in range(R_GROUP):
                xe = plsc.load_gather(xb, [lane_row[rr], r])
                plsc.addupdate_scatter(ob, [lane_row[rr], cc], xe * vv)

    def cleanup(ob):
        @plsc.parallel_loop(0, N_CHUNK, unroll=4)
        def chunk_body(j):
            cc = cols_v[pl.ds(j * L, L)]
            for rr in range(R_GROUP):
                plsc.store_scatter(ob, [lane_row[rr], cc], zero)

    in_bufs = ((xb0, sin0), (xb1, sin1), (xb2, sin2), (xb3, sin3))
    out_bufs = ((ob0, sout0), (ob1, sout1))

    for b, (xb, sin) in enumerate(in_bufs):
        start_in(b, xb, sin)

    @plsc.parallel_loop(0, N_CHUNK, unroll=2)
    def permute_body(j):
        sl = pl.ds(j * L, L)
        p = pos_u[sl]
        slot = (p % N_CHUNK) * L + p // N_CHUNK
        plsc.store_scatter(rows_v, [slot], rows_u[sl])
        plsc.store_scatter(cols_v, [slot], cols_u[sl])
        plsc.store_scatter(vals_v, [slot], vals_u[sl])

    @plsc.parallel_loop(0, OUTPUT_SIZE // L, unroll=4)
    def zero_body(j):
        for ob, _ in out_bufs:
            for rr in range(R_GROUP):
                ob[rr, pl.ds(j * L, L)] = zero

    def ring_body(h, carry):
        for q in range(N_IN_BUF):
            g = N_IN_BUF * h + q
            xb, sin = in_bufs[q]
            ob, sout = out_bufs[q % N_OUT_BUF]

            @pl.when(g >= N_OUT_BUF)
            def _():
                wait_out(ob, sout)
                cleanup(ob)

            wait_in(xb, sin)
            accumulate(xb, ob)
            start_out(g, ob, sout)

            @pl.when(g + N_IN_BUF <= N_GROUP - 1)
            def _():
                start_in(g + N_IN_BUF, xb, sin)

        return carry

    lax.fori_loop(0, N_GROUP // N_IN_BUF, ring_body, 0)

    for ob, sout in out_bufs:
        wait_out(ob, sout)


@jax.jit
def _sparse_mm_sc(x, rows, cols, vals, pos):
    kern = pl.kernel(
        _sc_body,
        out_type=jax.ShapeDtypeStruct((BATCH, OUTPUT_SIZE), jnp.float32),
        mesh=plsc.VectorSubcoreMesh(core_axis_name="c", subcore_axis_name="s"),
        scratch_types=[
            pltpu.VMEM((N_POINTS,), jnp.int32),
            pltpu.VMEM((N_POINTS,), jnp.int32),
            pltpu.VMEM((N_POINTS,), jnp.float32),
            pltpu.VMEM((N_POINTS,), jnp.int32),
            pltpu.VMEM((N_POINTS,), jnp.int32),
            pltpu.VMEM((N_POINTS,), jnp.int32),
            pltpu.VMEM((N_POINTS,), jnp.float32),
            *[pltpu.VMEM((R_GROUP, INPUT_SIZE), jnp.float32) for _ in range(N_IN_BUF)],
            *[pltpu.VMEM((R_GROUP, OUTPUT_SIZE), jnp.float32) for _ in range(N_OUT_BUF)],
            *[pltpu.SemaphoreType.DMA for _ in range(N_IN_BUF + N_OUT_BUF)],
        ],
        compiler_params=pltpu.CompilerParams(needs_layout_passes=False),
    )
    return kern(x, rows, cols, vals, pos)


def kernel(x, D, sigma, v):
    indices, values = _sample_weight(D, sigma, v)
    rows = indices[:, 0]
    cols = indices[:, 1]

    iota = jnp.arange(N_POINTS, dtype=jnp.int32)
    key = cols * jnp.int32(2048) + iota
    pos = jnp.sum((key[None, :] < key[:, None]).astype(jnp.int32), axis=1)

    return _sparse_mm_sc(x, rows, cols, values, pos)
